# baseline (speedup 1.0000x reference)
import functools

import jax
import jax.numpy as jnp
from jax import lax
from jax.experimental import pallas as pl
from jax.experimental.pallas import tpu as pltpu
from jax.experimental.pallas import tpu_sc as plsc

_NC = 2
_NS = 16
_NW = _NC * _NS
_CHUNK = 128
_EMB = 64


@functools.lru_cache(maxsize=None)
def _make_gather(B, D, idx_dtype):
    k_per_w = B // (_NW * _CHUNK)
    b_per_w = k_per_w * _CHUNK
    mesh = plsc.VectorSubcoreMesh(
        core_axis_name="c", subcore_axis_name="s",
        num_cores=_NC, num_subcores=_NS)

    @functools.partial(
        pl.kernel,
        mesh=mesh,
        compiler_params=pltpu.CompilerParams(use_tc_tiling_on_sc=False),
        out_type=[jax.ShapeDtypeStruct((B, D), jnp.float32),
                  jax.ShapeDtypeStruct((B, D), jnp.float32)],
        scratch_types=[
            pltpu.VMEM((k_per_w, _CHUNK), idx_dtype),
            pltpu.VMEM((k_per_w, _CHUNK), idx_dtype),
            pltpu.VMEM((b_per_w, D), jnp.float32),
            pltpu.VMEM((b_per_w, D), jnp.float32),
            pltpu.SemaphoreType.DMA,
            pltpu.SemaphoreType.DMA,
        ],
    )
    def gather(uid_hbm, iid_hbm, ut_hbm, it_hbm, ue_hbm, ie_hbm,
               uidx_v, iidx_v, urow_v, irow_v, sem_u, sem_i):
        wid = lax.axis_index("s") * _NC + lax.axis_index("c")
        base = wid * b_per_w
        pltpu.sync_copy(uid_hbm.at[wid], uidx_v)
        pltpu.sync_copy(iid_hbm.at[wid], iidx_v)
        copies = []
        for j in range(k_per_w):
            dst = pl.ds(j * _CHUNK, _CHUNK)
            copies.append(pltpu.async_copy(
                ut_hbm.at[uidx_v.at[j]], urow_v.at[dst], sem_u))
            copies.append(pltpu.async_copy(
                it_hbm.at[iidx_v.at[j]], irow_v.at[dst], sem_i))
        for c in copies:
            c.wait()
        pltpu.sync_copy(urow_v, ue_hbm.at[pl.ds(base, b_per_w)])
        pltpu.sync_copy(irow_v, ie_hbm.at[pl.ds(base, b_per_w)])

    return gather


def _layernorm(x, g, b, eps=1e-5):
    mu = jnp.mean(x, axis=-1, keepdims=True)
    xc = x - mu
    var = jnp.mean(xc * xc, axis=-1, keepdims=True)
    return xc / jnp.sqrt(var + eps) * g + b


def _gelu(x):
    return 0.5 * x * (1.0 + lax.erf(x * 0.7071067811865476))


def _mlp_body(ue, ie, w0u, w0i, b0, g0, be0, w1, b1, g1, be1,
              w2, b2, g2, be2, wo, bo, out):
    f32 = jnp.float32
    x = (jnp.dot(ue[...], w0u[...], preferred_element_type=f32)
         + jnp.dot(ie[...], w0i[...], preferred_element_type=f32)
         + b0[...])
    x = _gelu(_layernorm(x, g0[...], be0[...]))
    x = jnp.dot(x, w1[...], preferred_element_type=f32) + b1[...]
    x = _gelu(_layernorm(x, g1[...], be1[...]))
    x = jnp.dot(x, w2[...], preferred_element_type=f32) + b2[...]
    x = _gelu(_layernorm(x, g2[...], be2[...]))
    out[...] = jnp.sum(x * wo[...], axis=-1, keepdims=True) + bo[...]


@functools.lru_cache(maxsize=None)
def _make_mlp(B, tile, h0, h1, h2, interpret=False):
    grid = B // tile
    row = lambda i: (i, 0)
    rep = lambda i: (0, 0)
    in_specs = [
        pl.BlockSpec((tile, _EMB), row),
        pl.BlockSpec((tile, _EMB), row),
        pl.BlockSpec((_EMB, h0), rep),
        pl.BlockSpec((_EMB, h0), rep),
        pl.BlockSpec((1, h0), rep),
        pl.BlockSpec((1, h0), rep),
        pl.BlockSpec((1, h0), rep),
        pl.BlockSpec((h0, h1), rep),
        pl.BlockSpec((1, h1), rep),
        pl.BlockSpec((1, h1), rep),
        pl.BlockSpec((1, h1), rep),
        pl.BlockSpec((h1, h2), rep),
        pl.BlockSpec((1, h2), rep),
        pl.BlockSpec((1, h2), rep),
        pl.BlockSpec((1, h2), rep),
        pl.BlockSpec((1, h2), rep),
        pl.BlockSpec((1, 1), rep),
    ]
    return pl.pallas_call(
        _mlp_body,
        grid=(grid,),
        in_specs=in_specs,
        out_specs=pl.BlockSpec((tile, 1), row),
        out_shape=jax.ShapeDtypeStruct((B, 1), jnp.float32),
        interpret=interpret,
    )


def kernel(user_ids, item_ids, user_table, item_table,
           W0, b0, g0, beta0, W1, b1, g1, beta1, W2, b2, g2, beta2,
           W_out, b_out):
    B = user_ids.shape[0]
    D = user_table.shape[1]
    uid3 = user_ids.astype(jnp.int32).reshape(_NW, -1, _CHUNK)
    iid3 = item_ids.astype(jnp.int32).reshape(_NW, -1, _CHUNK)
    ue, ie = _make_gather(B, D, jnp.int32)(
        uid3, iid3, user_table, item_table)

    h0, h1, h2 = W0.shape[1], W1.shape[1], W2.shape[1]
    mlp = _make_mlp(B, 1024, h0, h1, h2)
    return mlp(ue, ie, W0[:D], W0[D:],
               b0.reshape(1, -1), g0.reshape(1, -1), beta0.reshape(1, -1),
               W1, b1.reshape(1, -1), g1.reshape(1, -1), beta1.reshape(1, -1),
               W2, b2.reshape(1, -1), g2.reshape(1, -1), beta2.reshape(1, -1),
               W_out.reshape(1, -1), b_out.reshape(1, 1))


# speedup vs baseline: 1.5554x; 1.5554x over previous
import functools

import jax
import jax.numpy as jnp
from jax import lax
from jax.experimental import pallas as pl
from jax.experimental.pallas import tpu as pltpu
from jax.experimental.pallas import tpu_sc as plsc

_NC = 2
_NS = 16
_NW = _NC * _NS
_CHUNK = 128
_EMB = 64


@functools.lru_cache(maxsize=None)
def _make_gather(B, D, idx_dtype):
    b_per_w = B // _NW
    p_rows = 256
    n_pass = b_per_w // p_rows
    mesh = plsc.VectorSubcoreMesh(
        core_axis_name="c", subcore_axis_name="s",
        num_cores=_NC, num_subcores=_NS)

    @functools.partial(
        pl.kernel,
        mesh=mesh,
        compiler_params=pltpu.CompilerParams(needs_layout_passes=False),
        out_type=[jax.ShapeDtypeStruct((B, D), jnp.float32),
                  jax.ShapeDtypeStruct((B, D), jnp.float32)],
        scratch_types=[
            pltpu.VMEM((b_per_w,), idx_dtype),
            pltpu.VMEM((b_per_w,), idx_dtype),
            pltpu.VMEM((p_rows, D), jnp.float32),
            pltpu.VMEM((p_rows, D), jnp.float32),
            pltpu.SemaphoreType.DMA,
            pltpu.SemaphoreType.DMA,
        ],
    )
    def gather(uid_hbm, iid_hbm, ut_hbm, it_hbm, ue_hbm, ie_hbm,
               uidx_v, iidx_v, urow_v, irow_v, sem_u, sem_i):
        wid = lax.axis_index("s") * _NC + lax.axis_index("c")
        base = wid * b_per_w
        pltpu.sync_copy(uid_hbm.at[pl.ds(base, b_per_w)], uidx_v)
        pltpu.sync_copy(iid_hbm.at[pl.ds(base, b_per_w)], iidx_v)
        lane = lax.iota(jnp.int32, 16)

        for p in range(n_pass):
            def fire(i, _):
                g = (i // 16) * 16
                k = i - g
                uvec = uidx_v[pl.ds(p * p_rows + g, 16)]
                ivec = iidx_v[pl.ds(p * p_rows + g, 16)]
                us = jnp.sum(jnp.where(lane == k, uvec, 0))
                vs = jnp.sum(jnp.where(lane == k, ivec, 0))
                pltpu.async_copy(ut_hbm.at[pl.ds(us, 1)],
                                 urow_v.at[pl.ds(i, 1)], sem_u)
                pltpu.async_copy(it_hbm.at[pl.ds(vs, 1)],
                                 irow_v.at[pl.ds(i, 1)], sem_i)
                return _

            lax.fori_loop(0, p_rows, fire, 0)
            pltpu.make_async_copy(
                ut_hbm.at[pl.ds(0, p_rows)], urow_v, sem_u).wait()
            pltpu.make_async_copy(
                it_hbm.at[pl.ds(0, p_rows)], irow_v, sem_i).wait()
            dst = pl.ds(base + p * p_rows, p_rows)
            pltpu.sync_copy(urow_v, ue_hbm.at[dst])
            pltpu.sync_copy(irow_v, ie_hbm.at[dst])

    return gather


def _layernorm(x, g, b, eps=1e-5):
    mu = jnp.mean(x, axis=-1, keepdims=True)
    xc = x - mu
    var = jnp.mean(xc * xc, axis=-1, keepdims=True)
    return xc / jnp.sqrt(var + eps) * g + b


def _gelu(x):
    return 0.5 * x * (1.0 + lax.erf(x * 0.7071067811865476))


def _mlp_body(ue, ie, w0u, w0i, b0, g0, be0, w1, b1, g1, be1,
              w2, b2, g2, be2, wo, bo, out):
    f32 = jnp.float32
    x = (jnp.dot(ue[...], w0u[...], preferred_element_type=f32)
         + jnp.dot(ie[...], w0i[...], preferred_element_type=f32)
         + b0[...])
    x = _gelu(_layernorm(x, g0[...], be0[...]))
    x = jnp.dot(x, w1[...], preferred_element_type=f32) + b1[...]
    x = _gelu(_layernorm(x, g1[...], be1[...]))
    x = jnp.dot(x, w2[...], preferred_element_type=f32) + b2[...]
    x = _gelu(_layernorm(x, g2[...], be2[...]))
    out[...] = jnp.sum(x * wo[...], axis=-1, keepdims=True) + bo[...]


@functools.lru_cache(maxsize=None)
def _make_mlp(B, tile, h0, h1, h2, interpret=False):
    grid = B // tile
    row = lambda i: (i, 0)
    rep = lambda i: (0, 0)
    in_specs = [
        pl.BlockSpec((tile, _EMB), row),
        pl.BlockSpec((tile, _EMB), row),
        pl.BlockSpec((_EMB, h0), rep),
        pl.BlockSpec((_EMB, h0), rep),
        pl.BlockSpec((1, h0), rep),
        pl.BlockSpec((1, h0), rep),
        pl.BlockSpec((1, h0), rep),
        pl.BlockSpec((h0, h1), rep),
        pl.BlockSpec((1, h1), rep),
        pl.BlockSpec((1, h1), rep),
        pl.BlockSpec((1, h1), rep),
        pl.BlockSpec((h1, h2), rep),
        pl.BlockSpec((1, h2), rep),
        pl.BlockSpec((1, h2), rep),
        pl.BlockSpec((1, h2), rep),
        pl.BlockSpec((1, h2), rep),
        pl.BlockSpec((1, 1), rep),
    ]
    return pl.pallas_call(
        _mlp_body,
        grid=(grid,),
        in_specs=in_specs,
        out_specs=pl.BlockSpec((tile, 1), row),
        out_shape=jax.ShapeDtypeStruct((B, 1), jnp.float32),
        interpret=interpret,
    )


def kernel(user_ids, item_ids, user_table, item_table,
           W0, b0, g0, beta0, W1, b1, g1, beta1, W2, b2, g2, beta2,
           W_out, b_out):
    B = user_ids.shape[0]
    D = user_table.shape[1]
    ue, ie = _make_gather(B, D, jnp.int32)(
        user_ids.astype(jnp.int32), item_ids.astype(jnp.int32),
        user_table, item_table)

    h0, h1, h2 = W0.shape[1], W1.shape[1], W2.shape[1]
    mlp = _make_mlp(B, 1024, h0, h1, h2)
    return mlp(ue, ie, W0[:D], W0[D:],
               b0.reshape(1, -1), g0.reshape(1, -1), beta0.reshape(1, -1),
               W1, b1.reshape(1, -1), g1.reshape(1, -1), beta1.reshape(1, -1),
               W2, b2.reshape(1, -1), g2.reshape(1, -1), beta2.reshape(1, -1),
               W_out.reshape(1, -1), b_out.reshape(1, 1))
